# baseline (speedup 1.0000x reference)
import functools

import jax
import jax.numpy as jnp
from jax import lax
from jax.experimental import pallas as pl
from jax.experimental.pallas import tpu as pltpu
from jax.experimental.pallas import tpu_sc as plsc

_PAIRS = ((0, 0), (1, 1), (2, 2), (3, 3),
          (0, 1), (0, 2), (0, 3), (1, 2), (1, 3), (2, 3))
_NWORK = 32


def _bf16_round(x):
    b = plsc.bitcast(x, jnp.int32)
    r = b + 0x8000 + ((b >> 16) & 1)
    return plsc.bitcast(r & jnp.int32(-65536), jnp.float32)


def _make_sc_kernel(n_pad, e, chunk, nch):
    w_per = e // _NWORK
    cpad = ((chunk + 15) // 16) * 16
    ngrp = cpad // 16
    nps = n_pad // 16
    mesh = plsc.VectorSubcoreMesh(core_axis_name="c", subcore_axis_name="s")

    @functools.partial(
        pl.kernel,
        mesh=mesh,
        compiler_params=pltpu.CompilerParams(use_tc_tiling_on_sc=False,
                                             needs_layout_passes=False),
        out_type=jax.ShapeDtypeStruct((2, n_pad, 16), jnp.float32),
        scratch_types=[
            pltpu.VMEM((cpad * 4,), jnp.float32),
            pltpu.VMEM((cpad * 12,), jnp.float32),
            pltpu.VMEM((chunk,), jnp.int32),
            pltpu.VMEM((cpad, 16), jnp.float32),
            pltpu.VMEM((256,), jnp.float32),
            pltpu.VMEM((nps, 16), jnp.float32),
            pltpu.VMEM_SHARED((n_pad, 16), jnp.float32),
        ],
    )
    def sc_kernel(gs_hbm, gv_hbm, pi_hbm, out_hbm,
                  gsb, gvb, ibuf, rowbuf, gbuf, zbuf, acc):
        cid = lax.axis_index("c")
        sid = lax.axis_index("s")
        wid = sid * 2 + cid
        base = wid * w_per
        zero16 = jnp.zeros((16,), jnp.float32)
        lane = lax.iota(jnp.int32, 16)

        def zrow(i, t):
            zbuf[i, :] = zero16
            return t

        lax.fori_loop(0, nps, zrow, 0)
        pltpu.sync_copy(zbuf, acc.at[pl.ds(sid * nps, nps)])
        for k in range(11, 16):
            gbuf[pl.ds(k * 16, 16)] = zero16
        plsc.subcore_barrier()

        for ch in range(nch):
            start = base + ch * chunk
            pltpu.sync_copy(gs_hbm.at[pl.ds(start * 4, chunk * 4)],
                            gsb.at[pl.ds(0, chunk * 4)])
            pltpu.sync_copy(gv_hbm.at[pl.ds(start * 12, chunk * 12)],
                            gvb.at[pl.ds(0, chunk * 12)])
            pltpu.sync_copy(pi_hbm.at[1, pl.ds(start, chunk)], ibuf)

            def group(g, t):
                o = g * 16
                cgs = [plsc.load_gather(gsb, [lane * 4 + (o * 4 + j)])
                       for j in range(4)]
                gvr = [_bf16_round(
                    plsc.load_gather(gvb, [lane * 12 + (o * 12 + j)]))
                    for j in range(12)]
                for k, (a, b) in enumerate(_PAIRS):
                    gbuf[pl.ds(k * 16, 16)] = (
                        gvr[a] * gvr[b] + gvr[4 + a] * gvr[4 + b]
                        + gvr[8 + a] * gvr[8 + b])
                gbuf[pl.ds(160, 16)] = cgs[0] + cgs[1] + cgs[2] + cgs[3]
                for i in range(16):
                    rowbuf[o + i, :] = plsc.load_gather(gbuf, [lane * 16 + i])
                return t

            lax.fori_loop(0, ngrp, group, 0)
            pltpu.sync_copy(rowbuf.at[pl.ds(0, chunk)],
                            acc.at[ibuf], add=True)

        plsc.subcore_barrier()
        pltpu.sync_copy(acc.at[pl.ds(sid * nps, nps)],
                        out_hbm.at[cid, pl.ds(sid * nps, nps)])

    return sc_kernel


def _sc_partials(gs_flat, gv_flat, pair_indices, n_pad, e):
    chunk = 1000
    nch = (e // _NWORK) // chunk
    return _make_sc_kernel(n_pad, e, chunk, nch)(gs_flat, gv_flat,
                                                 pair_indices)


def _dot(a, b):
    return jax.lax.dot(a, b, preferred_element_type=jnp.float32)


def _node_body(accp, emb, q, agh2d, w1a, w1b, w1c, b1, w2, b2,
               w3a, b3a, w3b, b3b, out_a, out_b):
    gs16 = accp[0, :, :] + accp[1, :, :]
    s = gs16[:, 10:11]
    e = emb[...]
    t = _dot(e, agh2d[...])
    t = t.astype(jnp.bfloat16).astype(jnp.float32)
    wts = (1.0, 1.0, 1.0, 1.0, 2.0, 2.0, 2.0, 2.0, 2.0, 2.0)
    vec = None
    for k, ((a, b), w) in enumerate(zip(_PAIRS, wts)):
        term = gs16[:, k:k + 1] * (t[:, 8 * a:8 * a + 8] * t[:, 8 * b:8 * b + 8])
        if w != 1.0:
            term = term * w
        vec = term if vec is None else vec + term
    h = (_dot(s * e, w1a[...])
         + _dot(vec, w1b[...])
         + (s * q[...]) * w1c[...] + b1[...])
    h = jax.nn.gelu(h)
    h = jax.nn.gelu(_dot(h, w2[...]) + b2[...])
    out_a[...] = _dot(h, w3a[...]) + b3a[...]
    out_b[...] = _dot(h, w3b[...]) + b3b[...]


def _node_call(accp, emb, q, agh2d, w1a, w1b, w1c, b1, w2, b2,
               w3a, b3a, w3b, b3b):
    n, f = emb.shape
    bn = 1000 if n % 1000 == 0 else n
    grid = (n // bn,)

    def row(shape):
        return pl.BlockSpec(shape, lambda i: (i, 0))

    def full(shape):
        return pl.BlockSpec(shape, lambda i: (0, 0))

    return pl.pallas_call(
        _node_body,
        grid=grid,
        in_specs=[
            pl.BlockSpec((2, bn, 16), lambda i: (0, i, 0)),
            row((bn, f)), row((bn, 1)),
            full(agh2d.shape), full(w1a.shape), full(w1b.shape),
            full(w1c.shape), full(b1.shape), full(w2.shape), full(b2.shape),
            full(w3a.shape), full(b3a.shape), full(w3b.shape), full(b3b.shape),
        ],
        out_specs=[row((bn, f)), row((bn, 8))],
        out_shape=[
            jax.ShapeDtypeStruct((n, f), jnp.float32),
            jax.ShapeDtypeStruct((n, 8), jnp.float32),
        ],
    )(accp, emb, q, agh2d, w1a, w1b, w1c, b1, w2, b2,
      w3a, b3a, w3b, b3b)


def kernel(atomic_embedding, partial_charges, pair_indices, gs, gv, agh,
           W1, b1, W2, b2, W3, b3):
    n, f = atomic_embedding.shape
    e, g = gs.shape
    v = agh.shape[2]

    n_pad = ((n + 127) // 128) * 128
    partials = _sc_partials(gs.reshape(e * g), gv.reshape(e * 3 * g),
                            pair_indices, n_pad, e)

    agh2d = agh.reshape(f, g * v)
    w1a = W1[:f]
    w1b = W1[f:f + v]
    w1c = W1[f + v:f + v + 1]
    w3a = W3[:, 2:]
    w3b = jnp.pad(W3[:, :2], ((0, 0), (0, 6)))
    b3a = b3[2:].reshape(1, -1)
    b3b = jnp.pad(b3[:2], (0, 6)).reshape(1, -1)

    out_a, out_b = _node_call(
        partials, atomic_embedding, partial_charges, agh2d,
        w1a, w1b, w1c, b1.reshape(1, -1), W2, b2.reshape(1, -1),
        w3a, b3a, w3b, b3b)

    return (out_a, out_b[:, 0:1], out_b[:, 1:2])


# speedup vs baseline: 2.1960x; 2.1960x over previous
import functools

import jax
import jax.numpy as jnp
from jax import lax
from jax.experimental import pallas as pl
from jax.experimental.pallas import tpu as pltpu
from jax.experimental.pallas import tpu_sc as plsc

_PAIRS = ((0, 0), (1, 1), (2, 2), (3, 3),
          (0, 1), (0, 2), (0, 3), (1, 2), (1, 3), (2, 3))
_NWORK = 32


def _bf16_round(x):
    b = plsc.bitcast(x, jnp.int32)
    r = b + 0x8000 + ((b >> 16) & 1)
    return plsc.bitcast(r & jnp.int32(-65536), jnp.float32)


def _make_sc_kernel(n_pad, e, chunk, nch):
    w_per = e // _NWORK
    cpad = ((chunk + 15) // 16) * 16
    ngrp = cpad // 16
    nps = n_pad // 16
    mesh = plsc.VectorSubcoreMesh(core_axis_name="c", subcore_axis_name="s")

    @functools.partial(
        pl.kernel,
        mesh=mesh,
        compiler_params=pltpu.CompilerParams(use_tc_tiling_on_sc=False,
                                             needs_layout_passes=False),
        out_type=jax.ShapeDtypeStruct((2, n_pad, 16), jnp.float32),
        scratch_types=[
            pltpu.VMEM((cpad, 4), jnp.float32),
            pltpu.VMEM((cpad, 12), jnp.float32),
            pltpu.VMEM((chunk,), jnp.int32),
            pltpu.VMEM((cpad, 16), jnp.float32),
            pltpu.VMEM((256,), jnp.float32),
            pltpu.VMEM((nps, 16), jnp.float32),
            pltpu.VMEM_SHARED((n_pad, 16), jnp.float32),
        ],
    )
    def sc_kernel(gs_hbm, gv_hbm, pi_hbm, out_hbm,
                  gsb, gvb, ibuf, rowbuf, gbuf, zbuf, acc):
        cid = lax.axis_index("c")
        sid = lax.axis_index("s")
        wid = sid * 2 + cid
        base = wid * w_per
        zero16 = jnp.zeros((16,), jnp.float32)
        lane = lax.iota(jnp.int32, 16)

        def zrow(i, t):
            zbuf[i, :] = zero16
            return t

        lax.fori_loop(0, nps, zrow, 0)
        pltpu.sync_copy(zbuf, acc.at[pl.ds(sid * nps, nps)])
        for k in range(11, 16):
            gbuf[pl.ds(k * 16, 16)] = zero16
        plsc.subcore_barrier()

        for ch in range(nch):
            start = base + ch * chunk
            pltpu.sync_copy(gs_hbm.at[pl.ds(start, chunk)],
                            gsb.at[pl.ds(0, chunk)])
            pltpu.sync_copy(gv_hbm.at[pl.ds(start, chunk)],
                            gvb.at[pl.ds(0, chunk)])
            pltpu.sync_copy(pi_hbm.at[1, pl.ds(start, chunk)], ibuf)

            def group(g, t):
                o = g * 16
                erow = lane + o
                cgs = [plsc.load_gather(gsb, [erow, jnp.full((16,), j, jnp.int32)])
                       for j in range(4)]
                gvr = [_bf16_round(
                    plsc.load_gather(gvb, [erow, jnp.full((16,), j, jnp.int32)]))
                    for j in range(12)]
                for k, (a, b) in enumerate(_PAIRS):
                    gbuf[pl.ds(k * 16, 16)] = (
                        gvr[a] * gvr[b] + gvr[4 + a] * gvr[4 + b]
                        + gvr[8 + a] * gvr[8 + b])
                gbuf[pl.ds(160, 16)] = cgs[0] + cgs[1] + cgs[2] + cgs[3]
                for i in range(16):
                    rowbuf[o + i, :] = plsc.load_gather(gbuf, [lane * 16 + i])
                return t

            lax.fori_loop(0, ngrp, group, 0)
            pltpu.sync_copy(rowbuf.at[pl.ds(0, chunk)],
                            acc.at[ibuf], add=True)

        plsc.subcore_barrier()
        pltpu.sync_copy(acc.at[pl.ds(sid * nps, nps)],
                        out_hbm.at[cid, pl.ds(sid * nps, nps)])

    return sc_kernel


def _sc_partials(gs2d, gv2d, pair_indices, n_pad, e):
    chunk = 1000
    nch = (e // _NWORK) // chunk
    return _make_sc_kernel(n_pad, e, chunk, nch)(gs2d, gv2d, pair_indices)


def _dot(a, b):
    return jax.lax.dot(a, b, preferred_element_type=jnp.float32)


def _node_body(accp, emb, q, agh2d, w1a, w1b, w1c, b1, w2, b2,
               w3a, b3a, w3b, b3b, out_a, out_b):
    gs16 = accp[0, :, :] + accp[1, :, :]
    s = gs16[:, 10:11]
    e = emb[...]
    t = _dot(e, agh2d[...])
    t = t.astype(jnp.bfloat16).astype(jnp.float32)
    wts = (1.0, 1.0, 1.0, 1.0, 2.0, 2.0, 2.0, 2.0, 2.0, 2.0)
    vec = None
    for k, ((a, b), w) in enumerate(zip(_PAIRS, wts)):
        term = gs16[:, k:k + 1] * (t[:, 8 * a:8 * a + 8] * t[:, 8 * b:8 * b + 8])
        if w != 1.0:
            term = term * w
        vec = term if vec is None else vec + term
    h = (_dot(s * e, w1a[...])
         + _dot(vec, w1b[...])
         + (s * q[...]) * w1c[...] + b1[...])
    h = jax.nn.gelu(h)
    h = jax.nn.gelu(_dot(h, w2[...]) + b2[...])
    out_a[...] = _dot(h, w3a[...]) + b3a[...]
    out_b[...] = _dot(h, w3b[...]) + b3b[...]


def _node_call(accp, emb, q, agh2d, w1a, w1b, w1c, b1, w2, b2,
               w3a, b3a, w3b, b3b):
    n, f = emb.shape
    bn = 1000 if n % 1000 == 0 else n
    grid = (n // bn,)

    def row(shape):
        return pl.BlockSpec(shape, lambda i: (i, 0))

    def full(shape):
        return pl.BlockSpec(shape, lambda i: (0, 0))

    return pl.pallas_call(
        _node_body,
        grid=grid,
        in_specs=[
            pl.BlockSpec((2, bn, 16), lambda i: (0, i, 0)),
            row((bn, f)), row((bn, 1)),
            full(agh2d.shape), full(w1a.shape), full(w1b.shape),
            full(w1c.shape), full(b1.shape), full(w2.shape), full(b2.shape),
            full(w3a.shape), full(b3a.shape), full(w3b.shape), full(b3b.shape),
        ],
        out_specs=[row((bn, f)), row((bn, 8))],
        out_shape=[
            jax.ShapeDtypeStruct((n, f), jnp.float32),
            jax.ShapeDtypeStruct((n, 8), jnp.float32),
        ],
    )(accp, emb, q, agh2d, w1a, w1b, w1c, b1, w2, b2,
      w3a, b3a, w3b, b3b)


def kernel(atomic_embedding, partial_charges, pair_indices, gs, gv, agh,
           W1, b1, W2, b2, W3, b3):
    n, f = atomic_embedding.shape
    e, g = gs.shape
    v = agh.shape[2]

    n_pad = ((n + 127) // 128) * 128
    partials = _sc_partials(gs, gv.reshape(e, 3 * g), pair_indices, n_pad, e)

    agh2d = agh.reshape(f, g * v)
    w1a = W1[:f]
    w1b = W1[f:f + v]
    w1c = W1[f + v:f + v + 1]
    w3a = W3[:, 2:]
    w3b = jnp.pad(W3[:, :2], ((0, 0), (0, 6)))
    b3a = b3[2:].reshape(1, -1)
    b3b = jnp.pad(b3[:2], (0, 6)).reshape(1, -1)

    out_a, out_b = _node_call(
        partials, atomic_embedding, partial_charges, agh2d,
        w1a, w1b, w1c, b1.reshape(1, -1), W2, b2.reshape(1, -1),
        w3a, b3a, w3b, b3b)

    return (out_a, out_b[:, 0:1], out_b[:, 1:2])
